# Initial kernel scaffold; baseline (speedup 1.0000x reference)
import jax
import jax.numpy as jnp
from jax.experimental import pallas as pl


def kernel(x, bases, comb, w_self, edge_index, edge_type, indices):
    raise NotImplementedError("write your pallas kernel here")


# speedup vs baseline: 3.3376x; 3.3376x over previous
import functools

import jax
import jax.numpy as jnp
from jax import lax
from jax.experimental import pallas as pl
from jax.experimental.pallas import tpu as pltpu
from jax.experimental.pallas import tpu_sc as plsc

N = 10000
E = 320000
R = 16
NB_BASES = 8
D = 128
L16 = 16

NC, NS = 2, 16
NW = NC * NS
C = 128
WCHUNKS = (-(-E // (C * NW)) + 7) // 8 * 8
NCHUNKS = WCHUNKS * NW
E_PAD = NCHUNKS * C
N_ACC = 10240
ROWS_PER_TILE = N_ACC // NS

@functools.lru_cache(maxsize=None)
def _sc_mesh():
    return plsc.VectorSubcoreMesh(
        core_axis_name="c", subcore_axis_name="s", num_cores=NC, num_subcores=NS)


BN = 1000
NBLK = N // BN


def _hr_body(comb_ref, bases_ref, h_ref, out_ref, w_scr):
    r = pl.program_id(0)
    nb = pl.program_id(1)

    @pl.when(nb == 0)
    def _():
        w = comb_ref[r, 0] * bases_ref[0]
        for b in range(1, NB_BASES):
            w += comb_ref[r, b] * bases_ref[b]
        w_scr[...] = w

    out_ref[...] = jnp.dot(h_ref[...], w_scr[...],
                           preferred_element_type=jnp.float32)


_hr_call = pl.pallas_call(
    _hr_body,
    grid=(R, NBLK),
    in_specs=[
        pl.BlockSpec(memory_space=pltpu.SMEM),
        pl.BlockSpec((NB_BASES, D, D), lambda r, nb: (0, 0, 0)),
        pl.BlockSpec((BN, D), lambda r, nb: (nb, 0)),
    ],
    out_specs=pl.BlockSpec((BN, D), lambda r, nb: (r * NBLK + nb, 0)),
    out_shape=jax.ShapeDtypeStruct((R * N, D), jnp.float32),
    scratch_shapes=[pltpu.VMEM((D, D), jnp.float32)],
)


GRP = 16
NGRP = WCHUNKS // GRP
ZR = 64


@functools.lru_cache(maxsize=None)
def _make_edge_kernel():
    scratch = [
        pltpu.VMEM((GRP, C), jnp.int32),
        pltpu.VMEM((GRP, C), jnp.int32),
        pltpu.VMEM((GRP, C), jnp.int32),
        pltpu.VMEM((C,), jnp.int32),
        pltpu.VMEM((C, D), jnp.float32),
        pltpu.VMEM((ZR, D), jnp.float32),
        pltpu.VMEM_SHARED((N_ACC, D), jnp.float32),
        pltpu.SemaphoreType.DMA,
    ]

    def body(hr, srcr, typr, dstr, part_o,
             src_v, typ_v, dst_v, gidx_v, rows_v, zero_v, acc_sh, sem):
        c = lax.axis_index("c")
        s = lax.axis_index("s")
        wid = c * NS + s

        def _zb(i, carry):
            for j in range(D // L16):
                zero_v[i, pl.ds(j * L16, L16)] = jnp.zeros((L16,), jnp.float32)
            return carry
        lax.fori_loop(0, ZR, _zb, 0)

        base_row = s * ROWS_PER_TILE
        for k in range(ROWS_PER_TILE // ZR):
            pltpu.sync_copy(zero_v, acc_sh.at[pl.ds(base_row + k * ZR, ZR)])
        plsc.subcore_barrier()

        def _grp(g, carry):
            r0 = pl.multiple_of(wid * WCHUNKS + g * GRP, GRP)
            pltpu.sync_copy(srcr.at[pl.ds(r0, GRP)], src_v)
            pltpu.sync_copy(typr.at[pl.ds(r0, GRP)], typ_v)
            pltpu.sync_copy(dstr.at[pl.ds(r0, GRP)], dst_v)

            def _eb(k, carry2):
                for j in range(C // L16):
                    sl = pl.ds(j * L16, L16)
                    gidx_v[sl] = typ_v[k, sl] * N + src_v[k, sl]
                pltpu.async_copy(hr.at[gidx_v], rows_v, sem).wait()
                pltpu.sync_copy(rows_v, acc_sh.at[dst_v.at[k]], add=True)
                return carry2
            lax.fori_loop(0, GRP, _eb, 0)
            return carry
        lax.fori_loop(0, NGRP, _grp, 0)
        plsc.subcore_barrier()

        pltpu.sync_copy(acc_sh.at[pl.ds(base_row, ROWS_PER_TILE)],
                        part_o.at[c, pl.ds(base_row, ROWS_PER_TILE)])

    return pl.kernel(body,
                     out_type=jax.ShapeDtypeStruct((NC, N_ACC, D), jnp.float32),
                     mesh=_sc_mesh(), scratch_types=scratch)


@functools.lru_cache(maxsize=None)
def _make_cnt_kernel():
    scratch = [
        pltpu.VMEM((GRP, C), jnp.int32),
        pltpu.VMEM((C, D), jnp.float32),
        pltpu.VMEM((ZR, D), jnp.float32),
        pltpu.VMEM_SHARED((N_ACC, D), jnp.float32),
    ]

    def body(dstr, cnt_o, dst_v, ones_v, zero16_v, cnt_sh):
        c = lax.axis_index("c")
        s = lax.axis_index("s")
        wid = c * NS + s

        def _ob(i, carry):
            for j in range(D // L16):
                sl = pl.ds(j * L16, L16)
                ones_v[i % C, sl] = jnp.ones((L16,), jnp.float32)
                zero16_v[i % ZR, sl] = jnp.zeros((L16,), jnp.float32)
            return carry
        lax.fori_loop(0, C, _ob, 0)

        base_row = s * ROWS_PER_TILE
        for k in range(ROWS_PER_TILE // ZR):
            pltpu.sync_copy(zero16_v, cnt_sh.at[pl.ds(base_row + k * ZR, ZR)])
        plsc.subcore_barrier()

        def _grp(g, carry):
            r0 = pl.multiple_of(wid * WCHUNKS + g * GRP, GRP)
            pltpu.sync_copy(dstr.at[pl.ds(r0, GRP)], dst_v)

            def _eb(k, carry2):
                pltpu.sync_copy(ones_v, cnt_sh.at[dst_v.at[k]], add=True)
                return carry2
            lax.fori_loop(0, GRP, _eb, 0)
            return carry
        lax.fori_loop(0, NGRP, _grp, 0)
        plsc.subcore_barrier()

        pltpu.sync_copy(cnt_sh.at[pl.ds(base_row, ROWS_PER_TILE)],
                        cnt_o.at[c, pl.ds(base_row, ROWS_PER_TILE)])

    return pl.kernel(
        body,
        out_type=jax.ShapeDtypeStruct((NC, N_ACC, D), jnp.float32),
        mesh=_sc_mesh(), scratch_types=scratch)


def _upd_body(cnt_ref, p_ref, h_ref, ws_ref, out_ref):
    deg = cnt_ref[0, :, 0:1] + cnt_ref[1, :, 0:1]
    inv = 1.0 / jnp.maximum(deg, 1.0)
    agg = (p_ref[0] + p_ref[1]) * inv
    out_ref[...] = jnp.maximum(
        agg + jnp.dot(h_ref[...], ws_ref[...],
                      preferred_element_type=jnp.float32), 0.0)


_upd_call = pl.pallas_call(
    _upd_body,
    grid=(NBLK,),
    in_specs=[
        pl.BlockSpec((NC, BN, D), lambda nb: (0, nb, 0)),
        pl.BlockSpec((NC, BN, D), lambda nb: (0, nb, 0)),
        pl.BlockSpec((BN, D), lambda nb: (nb, 0)),
        pl.BlockSpec((D, D), lambda nb: (0, 0)),
    ],
    out_specs=pl.BlockSpec((BN, D), lambda nb: (nb, 0)),
    out_shape=jax.ShapeDtypeStruct((N, D), jnp.float32),
)


NG = 4096
GPW = NG // NW


def _gather_body(h2, idx_hbm, out_hbm, idx_v, rows_v, sem):
    c = lax.axis_index("c")
    s = lax.axis_index("s")
    wid = c * NS + s
    base = pl.multiple_of(wid * GPW, GPW)
    pltpu.sync_copy(idx_hbm.at[pl.ds(base, GPW)], idx_v)
    pltpu.async_copy(h2.at[idx_v], rows_v, sem).wait()
    pltpu.sync_copy(rows_v, out_hbm.at[pl.ds(base, GPW)])


@functools.lru_cache(maxsize=None)
def _make_gather_kernel():
    return pl.kernel(
        _gather_body,
        out_type=jax.ShapeDtypeStruct((NG, D), jnp.float32),
        mesh=_sc_mesh(),
        scratch_types=[
            pltpu.VMEM((GPW,), jnp.int32),
            pltpu.VMEM((GPW, D), jnp.float32),
            pltpu.SemaphoreType.DMA,
        ],
    )


def kernel(x, bases, comb, w_self, edge_index, edge_type, indices):
    src, dst = edge_index[0], edge_index[1]
    padn = E_PAD - E
    src_p = jnp.pad(src, (0, padn)).reshape(NCHUNKS, C)
    typ_p = jnp.pad(edge_type, (0, padn)).reshape(NCHUNKS, C)
    dst_p = jnp.pad(dst, (0, padn), constant_values=N).reshape(NCHUNKS, C)

    h = x
    cnt = _make_cnt_kernel()(dst_p)
    for layer in range(bases.shape[0]):
        hr = _hr_call(comb[layer], bases[layer], h)
        part = _make_edge_kernel()(hr, src_p, typ_p, dst_p)
        h = _upd_call(cnt, part, h, w_self[layer])
    return _make_gather_kernel()(h, indices)


# speedup vs baseline: 3.5293x; 1.0574x over previous
import functools

import jax
import jax.numpy as jnp
from jax import lax
from jax.experimental import pallas as pl
from jax.experimental.pallas import tpu as pltpu
from jax.experimental.pallas import tpu_sc as plsc

N = 10000
E = 320000
R = 16
NB_BASES = 8
D = 128
L16 = 16

NC, NS = 2, 16
NW = NC * NS
C = 128
WCHUNKS = (-(-E // (C * NW)) + 7) // 8 * 8
NCHUNKS = WCHUNKS * NW
E_PAD = NCHUNKS * C
N_ACC = 10240
ROWS_PER_TILE = N_ACC // NS

@functools.lru_cache(maxsize=None)
def _sc_mesh():
    return plsc.VectorSubcoreMesh(
        core_axis_name="c", subcore_axis_name="s", num_cores=NC, num_subcores=NS)


BN = 1000
NBLK = N // BN


def _hr_body(comb_ref, bases_ref, h_ref, out_ref, w_scr):
    r = pl.program_id(0)
    nb = pl.program_id(1)

    @pl.when(nb == 0)
    def _():
        w = comb_ref[r, 0] * bases_ref[0]
        for b in range(1, NB_BASES):
            w += comb_ref[r, b] * bases_ref[b]
        w_scr[...] = w

    out_ref[...] = jnp.dot(h_ref[...], w_scr[...],
                           preferred_element_type=jnp.float32)


_hr_call = pl.pallas_call(
    _hr_body,
    grid=(R, NBLK),
    in_specs=[
        pl.BlockSpec(memory_space=pltpu.SMEM),
        pl.BlockSpec((NB_BASES, D, D), lambda r, nb: (0, 0, 0)),
        pl.BlockSpec((BN, D), lambda r, nb: (nb, 0)),
    ],
    out_specs=pl.BlockSpec((BN, D), lambda r, nb: (r * NBLK + nb, 0)),
    out_shape=jax.ShapeDtypeStruct((R * N, D), jnp.float32),
    scratch_shapes=[pltpu.VMEM((D, D), jnp.float32)],
)


GRP = 16
NGRP = WCHUNKS // GRP
ZR = 64
HALF = WCHUNKS // 2


@functools.lru_cache(maxsize=None)
def _make_edge_kernel():
    scratch = [
        pltpu.VMEM((HALF, C), jnp.int32),
        pltpu.VMEM((HALF, C), jnp.int32),
        pltpu.VMEM((C, D), jnp.float32),
        pltpu.VMEM((C, D), jnp.float32),
        pltpu.VMEM_SHARED((N_ACC, D), jnp.float32),
        pltpu.SemaphoreType.DMA,
        pltpu.SemaphoreType.DMA,
    ]

    def body(gidxr, dstr, hr, part_o,
             gidx_v, dst_v, rows0, rows1, acc_sh, sem0, sem1):
        c = lax.axis_index("c")
        s = lax.axis_index("s")
        wid = c * NS + s

        def _zb(i, carry):
            for j in range(D // L16):
                rows0[i, pl.ds(j * L16, L16)] = jnp.zeros((L16,), jnp.float32)
            return carry
        lax.fori_loop(0, C, _zb, 0)
        base_row = s * ROWS_PER_TILE
        for k in range(ROWS_PER_TILE // C):
            pltpu.sync_copy(rows0, acc_sh.at[pl.ds(base_row + k * C, C)])
        plsc.subcore_barrier()

        def _half(h, carry):
            r0 = pl.multiple_of(wid * WCHUNKS + h * HALF, 8)
            pltpu.sync_copy(gidxr.at[pl.ds(r0, HALF)], gidx_v)
            pltpu.sync_copy(dstr.at[pl.ds(r0, HALF)], dst_v)
            pltpu.async_copy(hr.at[gidx_v.at[0]], rows0, sem0)

            def _pair(g, carry2):
                k0 = 2 * g
                pltpu.async_copy(hr.at[gidx_v.at[k0 + 1]], rows1, sem1)
                pltpu.make_async_copy(hr.at[gidx_v.at[k0]], rows0, sem0).wait()
                pltpu.sync_copy(rows0, acc_sh.at[dst_v.at[k0]], add=True)

                @pl.when(g < HALF // 2 - 1)
                def _():
                    pltpu.async_copy(hr.at[gidx_v.at[k0 + 2]], rows0, sem0)
                pltpu.make_async_copy(hr.at[gidx_v.at[k0 + 1]], rows1,
                                      sem1).wait()
                pltpu.sync_copy(rows1, acc_sh.at[dst_v.at[k0 + 1]], add=True)
                return carry2
            lax.fori_loop(0, HALF // 2, _pair, 0)
            return carry
        lax.fori_loop(0, 2, _half, 0)
        plsc.subcore_barrier()

        pltpu.sync_copy(acc_sh.at[pl.ds(base_row, ROWS_PER_TILE)],
                        part_o.at[c, pl.ds(base_row, ROWS_PER_TILE)])

    return pl.kernel(body,
                     out_type=jax.ShapeDtypeStruct((NC, N_ACC, D), jnp.float32),
                     mesh=_sc_mesh(), scratch_types=scratch)


def _gidx_body(src_ref, typ_ref, out_ref):
    out_ref[...] = typ_ref[...] * N + src_ref[...]


_gidx_call = pl.pallas_call(
    _gidx_body,
    grid=(8,),
    in_specs=[pl.BlockSpec((NCHUNKS // 8, C), lambda i: (i, 0)),
              pl.BlockSpec((NCHUNKS // 8, C), lambda i: (i, 0))],
    out_specs=pl.BlockSpec((NCHUNKS // 8, C), lambda i: (i, 0)),
    out_shape=jax.ShapeDtypeStruct((NCHUNKS, C), jnp.int32),
)


@functools.lru_cache(maxsize=None)
def _make_cnt_kernel():
    scratch = [
        pltpu.VMEM((GRP, C), jnp.int32),
        pltpu.VMEM((C, D), jnp.float32),
        pltpu.VMEM((ZR, D), jnp.float32),
        pltpu.VMEM_SHARED((N_ACC, D), jnp.float32),
    ]

    def body(dstr, cnt_o, dst_v, ones_v, zero16_v, cnt_sh):
        c = lax.axis_index("c")
        s = lax.axis_index("s")
        wid = c * NS + s

        def _ob(i, carry):
            for j in range(D // L16):
                sl = pl.ds(j * L16, L16)
                ones_v[i % C, sl] = jnp.ones((L16,), jnp.float32)
                zero16_v[i % ZR, sl] = jnp.zeros((L16,), jnp.float32)
            return carry
        lax.fori_loop(0, C, _ob, 0)

        base_row = s * ROWS_PER_TILE
        for k in range(ROWS_PER_TILE // ZR):
            pltpu.sync_copy(zero16_v, cnt_sh.at[pl.ds(base_row + k * ZR, ZR)])
        plsc.subcore_barrier()

        def _grp(g, carry):
            r0 = pl.multiple_of(wid * WCHUNKS + g * GRP, GRP)
            pltpu.sync_copy(dstr.at[pl.ds(r0, GRP)], dst_v)

            def _eb(k, carry2):
                pltpu.sync_copy(ones_v, cnt_sh.at[dst_v.at[k]], add=True)
                return carry2
            lax.fori_loop(0, GRP, _eb, 0)
            return carry
        lax.fori_loop(0, NGRP, _grp, 0)
        plsc.subcore_barrier()

        pltpu.sync_copy(cnt_sh.at[pl.ds(base_row, ROWS_PER_TILE)],
                        cnt_o.at[c, pl.ds(base_row, ROWS_PER_TILE)])

    return pl.kernel(
        body,
        out_type=jax.ShapeDtypeStruct((NC, N_ACC, D), jnp.float32),
        mesh=_sc_mesh(), scratch_types=scratch)


def _upd_body(cnt_ref, p_ref, h_ref, ws_ref, out_ref):
    deg = cnt_ref[0, :, 0:1] + cnt_ref[1, :, 0:1]
    inv = 1.0 / jnp.maximum(deg, 1.0)
    agg = (p_ref[0] + p_ref[1]) * inv
    out_ref[...] = jnp.maximum(
        agg + jnp.dot(h_ref[...], ws_ref[...],
                      preferred_element_type=jnp.float32), 0.0)


_upd_call = pl.pallas_call(
    _upd_body,
    grid=(NBLK,),
    in_specs=[
        pl.BlockSpec((NC, BN, D), lambda nb: (0, nb, 0)),
        pl.BlockSpec((NC, BN, D), lambda nb: (0, nb, 0)),
        pl.BlockSpec((BN, D), lambda nb: (nb, 0)),
        pl.BlockSpec((D, D), lambda nb: (0, 0)),
    ],
    out_specs=pl.BlockSpec((BN, D), lambda nb: (nb, 0)),
    out_shape=jax.ShapeDtypeStruct((N, D), jnp.float32),
)


NG = 4096
GPW = NG // NW


def _gather_body(h2, idx_hbm, out_hbm, idx_v, rows_v, sem):
    c = lax.axis_index("c")
    s = lax.axis_index("s")
    wid = c * NS + s
    base = pl.multiple_of(wid * GPW, GPW)
    pltpu.sync_copy(idx_hbm.at[pl.ds(base, GPW)], idx_v)
    pltpu.async_copy(h2.at[idx_v], rows_v, sem).wait()
    pltpu.sync_copy(rows_v, out_hbm.at[pl.ds(base, GPW)])


@functools.lru_cache(maxsize=None)
def _make_gather_kernel():
    return pl.kernel(
        _gather_body,
        out_type=jax.ShapeDtypeStruct((NG, D), jnp.float32),
        mesh=_sc_mesh(),
        scratch_types=[
            pltpu.VMEM((GPW,), jnp.int32),
            pltpu.VMEM((GPW, D), jnp.float32),
            pltpu.SemaphoreType.DMA,
        ],
    )


def kernel(x, bases, comb, w_self, edge_index, edge_type, indices):
    src, dst = edge_index[0], edge_index[1]
    padn = E_PAD - E
    src_p = jnp.pad(src, (0, padn)).reshape(NCHUNKS, C)
    typ_p = jnp.pad(edge_type, (0, padn)).reshape(NCHUNKS, C)
    dst_p = jnp.pad(dst, (0, padn), constant_values=N).reshape(NCHUNKS, C)

    h = x
    gidx_p = _gidx_call(src_p, typ_p)
    cnt = _make_cnt_kernel()(dst_p)
    for layer in range(bases.shape[0]):
        hr = _hr_call(comb[layer], bases[layer], h)
        part = _make_edge_kernel()(gidx_p, dst_p, hr)
        h = _upd_call(cnt, part, h, w_self[layer])
    return _make_gather_kernel()(h, indices)


# speedup vs baseline: 8.1696x; 2.3148x over previous
import functools

import jax
import jax.numpy as jnp
from jax import lax
from jax.experimental import pallas as pl
from jax.experimental.pallas import tpu as pltpu
from jax.experimental.pallas import tpu_sc as plsc

N = 10000
E = 320000
R = 16
NB_BASES = 8
D = 128
L16 = 16

NC, NS = 2, 16
NW = NC * NS
C = 128
WCHUNKS = (-(-E // (C * NW)) + 7) // 8 * 8
NCHUNKS = WCHUNKS * NW
E_PAD = NCHUNKS * C
N_ACC = 10240
ROWS_PER_TILE = N_ACC // NS

@functools.lru_cache(maxsize=None)
def _sc_mesh():
    return plsc.VectorSubcoreMesh(
        core_axis_name="c", subcore_axis_name="s", num_cores=NC, num_subcores=NS)


BN = 1000
NBLK = N // BN


def _hr_body(comb_ref, bases_ref, h_ref, out_ref, w_scr):
    r = pl.program_id(0)
    nb = pl.program_id(1)

    @pl.when(nb == 0)
    def _():
        w = comb_ref[r, 0] * bases_ref[0]
        for b in range(1, NB_BASES):
            w += comb_ref[r, b] * bases_ref[b]
        w_scr[...] = w

    out_ref[...] = jnp.dot(h_ref[...], w_scr[...],
                           preferred_element_type=jnp.float32)


_hr_call = pl.pallas_call(
    _hr_body,
    grid=(R, NBLK),
    in_specs=[
        pl.BlockSpec(memory_space=pltpu.SMEM),
        pl.BlockSpec((NB_BASES, D, D), lambda r, nb: (0, 0, 0)),
        pl.BlockSpec((BN, D), lambda r, nb: (nb, 0)),
    ],
    out_specs=pl.BlockSpec((BN, D), lambda r, nb: (r * NBLK + nb, 0)),
    out_shape=jax.ShapeDtypeStruct((R * N, D), jnp.float32),
    scratch_shapes=[pltpu.VMEM((D, D), jnp.float32)],
)


GRP = 16
NGRP = WCHUNKS // GRP
ZR = 64
HALF = WCHUNKS // 2


@functools.lru_cache(maxsize=None)
def _make_edge_kernel():
    scratch = [
        pltpu.VMEM((HALF, C), jnp.int32),
        pltpu.VMEM((HALF, C), jnp.int32),
        pltpu.VMEM((C, D), jnp.float32),
        pltpu.VMEM((C, D), jnp.float32),
        pltpu.VMEM_SHARED((N_ACC, D), jnp.float32),
        pltpu.SemaphoreType.DMA,
        pltpu.SemaphoreType.DMA,
    ]

    def body(gidxr, dstr, hr, part_o,
             gidx_v, dst_v, rows0, rows1, acc_sh, sem0, sem1):
        c = lax.axis_index("c")
        s = lax.axis_index("s")
        wid = c * NS + s

        def _zb(i, carry):
            for j in range(D // L16):
                rows0[i, pl.ds(j * L16, L16)] = jnp.zeros((L16,), jnp.float32)
            return carry
        lax.fori_loop(0, C, _zb, 0)
        base_row = s * ROWS_PER_TILE
        for k in range(ROWS_PER_TILE // C):
            pltpu.sync_copy(rows0, acc_sh.at[pl.ds(base_row + k * C, C)])
        plsc.subcore_barrier()

        def _half(h, carry):
            r0 = pl.multiple_of(wid * WCHUNKS + h * HALF, 8)
            pltpu.sync_copy(gidxr.at[pl.ds(r0, HALF)], gidx_v)
            pltpu.sync_copy(dstr.at[pl.ds(r0, HALF)], dst_v)
            pltpu.async_copy(hr.at[gidx_v.at[0]], rows0, sem0)

            def _pair(g, carry2):
                k0 = 2 * g
                pltpu.async_copy(hr.at[gidx_v.at[k0 + 1]], rows1, sem1)
                pltpu.make_async_copy(hr.at[gidx_v.at[k0]], rows0, sem0).wait()
                pltpu.sync_copy(rows0, acc_sh.at[dst_v.at[k0]], add=True)

                @pl.when(g < HALF // 2 - 1)
                def _():
                    pltpu.async_copy(hr.at[gidx_v.at[k0 + 2]], rows0, sem0)
                pltpu.make_async_copy(hr.at[gidx_v.at[k0 + 1]], rows1,
                                      sem1).wait()
                pltpu.sync_copy(rows1, acc_sh.at[dst_v.at[k0 + 1]], add=True)
                return carry2
            lax.fori_loop(0, HALF // 2, _pair, 0)
            return carry
        lax.fori_loop(0, 2, _half, 0)
        plsc.subcore_barrier()

        pltpu.sync_copy(acc_sh.at[pl.ds(base_row, ROWS_PER_TILE)],
                        part_o.at[c, pl.ds(base_row, ROWS_PER_TILE)])

    return pl.kernel(body,
                     out_type=jax.ShapeDtypeStruct((NC, N_ACC, D), jnp.float32),
                     mesh=_sc_mesh(), scratch_types=scratch)


def _gidx_body(src_ref, typ_ref, out_ref):
    out_ref[...] = typ_ref[...] * N + src_ref[...]


_gidx_call = pl.pallas_call(
    _gidx_body,
    grid=(8,),
    in_specs=[pl.BlockSpec((NCHUNKS // 8, C), lambda i: (i, 0)),
              pl.BlockSpec((NCHUNKS // 8, C), lambda i: (i, 0))],
    out_specs=pl.BlockSpec((NCHUNKS // 8, C), lambda i: (i, 0)),
    out_shape=jax.ShapeDtypeStruct((NCHUNKS, C), jnp.int32),
)


@functools.lru_cache(maxsize=None)
def _make_cnt_kernel():
    scratch = [
        pltpu.VMEM((GRP, C), jnp.int32),
        pltpu.VMEM((C, D), jnp.float32),
        pltpu.VMEM((ZR, D), jnp.float32),
        pltpu.VMEM_SHARED((N_ACC, D), jnp.float32),
    ]

    def body(dstr, cnt_o, dst_v, ones_v, zero16_v, cnt_sh):
        c = lax.axis_index("c")
        s = lax.axis_index("s")
        wid = c * NS + s

        def _ob(i, carry):
            for j in range(D // L16):
                sl = pl.ds(j * L16, L16)
                ones_v[i % C, sl] = jnp.ones((L16,), jnp.float32)
                zero16_v[i % ZR, sl] = jnp.zeros((L16,), jnp.float32)
            return carry
        lax.fori_loop(0, C, _ob, 0)

        base_row = s * ROWS_PER_TILE
        for k in range(ROWS_PER_TILE // ZR):
            pltpu.sync_copy(zero16_v, cnt_sh.at[pl.ds(base_row + k * ZR, ZR)])
        plsc.subcore_barrier()

        def _grp(g, carry):
            r0 = pl.multiple_of(wid * WCHUNKS + g * GRP, GRP)
            pltpu.sync_copy(dstr.at[pl.ds(r0, GRP)], dst_v)

            def _eb(k, carry2):
                pltpu.sync_copy(ones_v, cnt_sh.at[dst_v.at[k]], add=True)
                return carry2
            lax.fori_loop(0, GRP, _eb, 0)
            return carry
        lax.fori_loop(0, NGRP, _grp, 0)
        plsc.subcore_barrier()

        pltpu.sync_copy(cnt_sh.at[pl.ds(base_row, ROWS_PER_TILE)],
                        cnt_o.at[c, pl.ds(base_row, ROWS_PER_TILE)])

    return pl.kernel(
        body,
        out_type=jax.ShapeDtypeStruct((NC, N_ACC, D), jnp.float32),
        mesh=_sc_mesh(), scratch_types=scratch)


def _upd_body(cnt_ref, p_ref, h_ref, ws_ref, out_ref):
    deg = cnt_ref[0, :, 0:1] + cnt_ref[1, :, 0:1]
    inv = 1.0 / jnp.maximum(deg, 1.0)
    agg = (p_ref[0] + p_ref[1]) * inv
    out_ref[...] = jnp.maximum(
        agg + jnp.dot(h_ref[...], ws_ref[...],
                      preferred_element_type=jnp.float32), 0.0)


_upd_call = pl.pallas_call(
    _upd_body,
    grid=(NBLK,),
    in_specs=[
        pl.BlockSpec((NC, BN, D), lambda nb: (0, nb, 0)),
        pl.BlockSpec((NC, BN, D), lambda nb: (0, nb, 0)),
        pl.BlockSpec((BN, D), lambda nb: (nb, 0)),
        pl.BlockSpec((D, D), lambda nb: (0, 0)),
    ],
    out_specs=pl.BlockSpec((BN, D), lambda nb: (nb, 0)),
    out_shape=jax.ShapeDtypeStruct((N, D), jnp.float32),
)


NG = 4096
GPW = NG // NW


def _gather_body(h2, idx_hbm, out_hbm, idx_v, rows_v, sem):
    c = lax.axis_index("c")
    s = lax.axis_index("s")
    wid = c * NS + s
    base = pl.multiple_of(wid * GPW, GPW)
    pltpu.sync_copy(idx_hbm.at[pl.ds(base, GPW)], idx_v)
    pltpu.async_copy(h2.at[idx_v], rows_v, sem).wait()
    pltpu.sync_copy(rows_v, out_hbm.at[pl.ds(base, GPW)])


@functools.lru_cache(maxsize=None)
def _make_gather_kernel():
    return pl.kernel(
        _gather_body,
        out_type=jax.ShapeDtypeStruct((NG, D), jnp.float32),
        mesh=_sc_mesh(),
        scratch_types=[
            pltpu.VMEM((GPW,), jnp.int32),
            pltpu.VMEM((GPW, D), jnp.float32),
            pltpu.SemaphoreType.DMA,
        ],
    )


def kernel(x, bases, comb, w_self, edge_index, edge_type, indices):
    src, dst = edge_index[0], edge_index[1]
    padn = E_PAD - E
    ar = jnp.arange(padn, dtype=jnp.int32)
    src_p = jnp.concatenate([src, (ar * 7919) % N]).reshape(NCHUNKS, C)
    typ_p = jnp.concatenate([edge_type, ar % R]).reshape(NCHUNKS, C)
    dst_p = jnp.concatenate([dst, N + ar % (N_ACC - N)]).reshape(NCHUNKS, C)

    h = x
    gidx_p = _gidx_call(src_p, typ_p)
    cnt = _make_cnt_kernel()(dst_p)
    for layer in range(bases.shape[0]):
        hr = _hr_call(comb[layer], bases[layer], h)
        part = _make_edge_kernel()(gidx_p, dst_p, hr)
        h = _upd_call(cnt, part, h, w_self[layer])
    return _make_gather_kernel()(h, indices)


# speedup vs baseline: 10.9064x; 1.3350x over previous
import functools

import jax
import jax.numpy as jnp
from jax import lax
from jax.experimental import pallas as pl
from jax.experimental.pallas import tpu as pltpu
from jax.experimental.pallas import tpu_sc as plsc

N = 10000
E = 320000
R = 16
NB_BASES = 8
D = 128
L16 = 16

NC, NS = 2, 16
NW = NC * NS
C = 128
WCHUNKS = (-(-E // (C * NW)) + 7) // 8 * 8
NCHUNKS = WCHUNKS * NW
E_PAD = NCHUNKS * C
N_ACC = 10240
ROWS_PER_TILE = N_ACC // NS

@functools.lru_cache(maxsize=None)
def _sc_mesh():
    return plsc.VectorSubcoreMesh(
        core_axis_name="c", subcore_axis_name="s", num_cores=NC, num_subcores=NS)


BN = 1000
NBLK = N // BN


BN_HR = 5000
NB_HR = N // BN_HR


def _hr_body(comb_ref, bases_ref, h_ref, out_ref, w_scr):
    r = pl.program_id(0)
    nb = pl.program_id(1)

    @pl.when(nb == 0)
    def _():
        w = comb_ref[r, 0] * bases_ref[0]
        for b in range(1, NB_BASES):
            w += comb_ref[r, b] * bases_ref[b]
        w_scr[...] = w.astype(jnp.bfloat16)

    out_ref[...] = jnp.dot(h_ref[...].astype(jnp.bfloat16), w_scr[...],
                           preferred_element_type=jnp.float32)


_hr_call = pl.pallas_call(
    _hr_body,
    grid=(R, NB_HR),
    in_specs=[
        pl.BlockSpec(memory_space=pltpu.SMEM),
        pl.BlockSpec((NB_BASES, D, D), lambda r, nb: (0, 0, 0)),
        pl.BlockSpec((BN_HR, D), lambda r, nb: (nb, 0)),
    ],
    out_specs=pl.BlockSpec((BN_HR, D), lambda r, nb: (r * NB_HR + nb, 0)),
    out_shape=jax.ShapeDtypeStruct((R * N, D), jnp.float32),
    scratch_shapes=[pltpu.VMEM((D, D), jnp.bfloat16)],
)


GRP = 16
NGRP = WCHUNKS // GRP
ZR = 64
HALF = WCHUNKS // 2


@functools.lru_cache(maxsize=None)
def _make_edge_kernel():
    scratch = [
        pltpu.VMEM((HALF, C), jnp.int32),
        pltpu.VMEM((HALF, C), jnp.int32),
        pltpu.VMEM((C, D), jnp.float32),
        pltpu.VMEM((C, D), jnp.float32),
        pltpu.VMEM_SHARED((N_ACC, D), jnp.float32),
        pltpu.SemaphoreType.DMA,
        pltpu.SemaphoreType.DMA,
    ]

    def body(gidxr, dstr, hr, part_o,
             gidx_v, dst_v, rows0, rows1, acc_sh, sem0, sem1):
        c = lax.axis_index("c")
        s = lax.axis_index("s")
        wid = c * NS + s

        def _zb(i, carry):
            for j in range(D // L16):
                rows0[i, pl.ds(j * L16, L16)] = jnp.zeros((L16,), jnp.float32)
            return carry
        lax.fori_loop(0, C, _zb, 0)
        base_row = s * ROWS_PER_TILE
        for k in range(ROWS_PER_TILE // C):
            pltpu.sync_copy(rows0, acc_sh.at[pl.ds(base_row + k * C, C)])
        plsc.subcore_barrier()

        def _half(h, carry):
            r0 = pl.multiple_of(wid * WCHUNKS + h * HALF, 8)
            pltpu.sync_copy(gidxr.at[pl.ds(r0, HALF)], gidx_v)
            pltpu.sync_copy(dstr.at[pl.ds(r0, HALF)], dst_v)
            pltpu.async_copy(hr.at[gidx_v.at[0]], rows0, sem0)

            def _pair(g, carry2):
                k0 = 2 * g
                pltpu.async_copy(hr.at[gidx_v.at[k0 + 1]], rows1, sem1)
                pltpu.make_async_copy(hr.at[gidx_v.at[k0]], rows0, sem0).wait()
                pltpu.sync_copy(rows0, acc_sh.at[dst_v.at[k0]], add=True)

                @pl.when(g < HALF // 2 - 1)
                def _():
                    pltpu.async_copy(hr.at[gidx_v.at[k0 + 2]], rows0, sem0)
                pltpu.make_async_copy(hr.at[gidx_v.at[k0 + 1]], rows1,
                                      sem1).wait()
                pltpu.sync_copy(rows1, acc_sh.at[dst_v.at[k0 + 1]], add=True)
                return carry2
            lax.fori_loop(0, HALF // 2, _pair, 0)
            return carry
        lax.fori_loop(0, 2, _half, 0)
        plsc.subcore_barrier()

        pltpu.sync_copy(acc_sh.at[pl.ds(base_row, ROWS_PER_TILE)],
                        part_o.at[c, pl.ds(base_row, ROWS_PER_TILE)])

    return pl.kernel(body,
                     out_type=jax.ShapeDtypeStruct((NC, N_ACC, D), jnp.float32),
                     mesh=_sc_mesh(), scratch_types=scratch)


def _gidx_body(src_ref, typ_ref, out_ref):
    out_ref[...] = typ_ref[...] * N + src_ref[...]


_gidx_call = pl.pallas_call(
    _gidx_body,
    grid=(8,),
    in_specs=[pl.BlockSpec((NCHUNKS // 8, C), lambda i: (i, 0)),
              pl.BlockSpec((NCHUNKS // 8, C), lambda i: (i, 0))],
    out_specs=pl.BlockSpec((NCHUNKS // 8, C), lambda i: (i, 0)),
    out_shape=jax.ShapeDtypeStruct((NCHUNKS, C), jnp.int32),
)


@functools.lru_cache(maxsize=None)
def _make_cnt_kernel():
    scratch = [
        pltpu.VMEM((GRP, C), jnp.int32),
        pltpu.VMEM((C, D), jnp.float32),
        pltpu.VMEM((ZR, D), jnp.float32),
        pltpu.VMEM_SHARED((N_ACC, D), jnp.float32),
    ]

    def body(dstr, cnt_o, dst_v, ones_v, zero16_v, cnt_sh):
        c = lax.axis_index("c")
        s = lax.axis_index("s")
        wid = c * NS + s

        def _ob(i, carry):
            for j in range(D // L16):
                sl = pl.ds(j * L16, L16)
                ones_v[i % C, sl] = jnp.ones((L16,), jnp.float32)
                zero16_v[i % ZR, sl] = jnp.zeros((L16,), jnp.float32)
            return carry
        lax.fori_loop(0, C, _ob, 0)

        base_row = s * ROWS_PER_TILE
        for k in range(ROWS_PER_TILE // ZR):
            pltpu.sync_copy(zero16_v, cnt_sh.at[pl.ds(base_row + k * ZR, ZR)])
        plsc.subcore_barrier()

        def _grp(g, carry):
            r0 = pl.multiple_of(wid * WCHUNKS + g * GRP, GRP)
            pltpu.sync_copy(dstr.at[pl.ds(r0, GRP)], dst_v)

            def _eb(k, carry2):
                pltpu.sync_copy(ones_v, cnt_sh.at[dst_v.at[k]], add=True)
                return carry2
            lax.fori_loop(0, GRP, _eb, 0)
            return carry
        lax.fori_loop(0, NGRP, _grp, 0)
        plsc.subcore_barrier()

        pltpu.sync_copy(cnt_sh.at[pl.ds(base_row, ROWS_PER_TILE)],
                        cnt_o.at[c, pl.ds(base_row, ROWS_PER_TILE)])

    return pl.kernel(
        body,
        out_type=jax.ShapeDtypeStruct((NC, N_ACC, D), jnp.float32),
        mesh=_sc_mesh(), scratch_types=scratch)


def _upd_body(cnt_ref, p_ref, h_ref, ws_ref, out_ref):
    deg = cnt_ref[0, :, 0:1] + cnt_ref[1, :, 0:1]
    inv = 1.0 / jnp.maximum(deg, 1.0)
    agg = (p_ref[0] + p_ref[1]) * inv
    out_ref[...] = jnp.maximum(
        agg + jnp.dot(h_ref[...], ws_ref[...],
                      preferred_element_type=jnp.float32), 0.0)


_upd_call = pl.pallas_call(
    _upd_body,
    grid=(NBLK,),
    in_specs=[
        pl.BlockSpec((NC, BN, D), lambda nb: (0, nb, 0)),
        pl.BlockSpec((NC, BN, D), lambda nb: (0, nb, 0)),
        pl.BlockSpec((BN, D), lambda nb: (nb, 0)),
        pl.BlockSpec((D, D), lambda nb: (0, 0)),
    ],
    out_specs=pl.BlockSpec((BN, D), lambda nb: (nb, 0)),
    out_shape=jax.ShapeDtypeStruct((N, D), jnp.float32),
)


NG = 4096
GPW = NG // NW


def _gather_body(h2, idx_hbm, out_hbm, idx_v, rows_v, sem):
    c = lax.axis_index("c")
    s = lax.axis_index("s")
    wid = c * NS + s
    base = pl.multiple_of(wid * GPW, GPW)
    pltpu.sync_copy(idx_hbm.at[pl.ds(base, GPW)], idx_v)
    pltpu.async_copy(h2.at[idx_v], rows_v, sem).wait()
    pltpu.sync_copy(rows_v, out_hbm.at[pl.ds(base, GPW)])


@functools.lru_cache(maxsize=None)
def _make_gather_kernel():
    return pl.kernel(
        _gather_body,
        out_type=jax.ShapeDtypeStruct((NG, D), jnp.float32),
        mesh=_sc_mesh(),
        scratch_types=[
            pltpu.VMEM((GPW,), jnp.int32),
            pltpu.VMEM((GPW, D), jnp.float32),
            pltpu.SemaphoreType.DMA,
        ],
    )


def kernel(x, bases, comb, w_self, edge_index, edge_type, indices):
    src, dst = edge_index[0], edge_index[1]
    padn = E_PAD - E
    ar = jnp.arange(padn, dtype=jnp.int32)
    src_p = jnp.concatenate([src, (ar * 7919) % N]).reshape(NCHUNKS, C)
    typ_p = jnp.concatenate([edge_type, ar % R]).reshape(NCHUNKS, C)
    dst_p = jnp.concatenate([dst, N + ar % (N_ACC - N)]).reshape(NCHUNKS, C)

    h = x
    gidx_p = _gidx_call(src_p, typ_p)
    cnt = _make_cnt_kernel()(dst_p)
    for layer in range(bases.shape[0]):
        hr = _hr_call(comb[layer], bases[layer], h)
        part = _make_edge_kernel()(gidx_p, dst_p, hr)
        h = _upd_call(cnt, part, h, w_self[layer])
    return _make_gather_kernel()(h, indices)
